# baseline (speedup 1.0000x reference)
import functools

import jax
import jax.numpy as jnp
from jax import lax
from jax.experimental import pallas as pl
from jax.experimental.pallas import tpu as pltpu
from jax.experimental.pallas import tpu_sc as plsc

VOCAB = 100000
EMBED_DIM = 64
CTX_LEN = 200
HIDDEN = 64
FLAT = CTX_LEN * EMBED_DIM

B_PER_W = 8


@functools.cache
def _make_sc_gather():
    info = plsc.get_sparse_core_info()
    nc = info.num_cores
    mesh = plsc.VectorSubcoreMesh(core_axis_name="c", subcore_axis_name="s")

    @functools.partial(
        pl.kernel,
        mesh=mesh,
        out_type=jax.ShapeDtypeStruct((1, FLAT), jnp.float32),
        scratch_types=[
            pltpu.VMEM((B_PER_W,), jnp.int32),
            pltpu.VMEM((B_PER_W, EMBED_DIM), jnp.float32),
            pltpu.SemaphoreType.DMA,
        ],
        compiler_params=pltpu.CompilerParams(use_tc_tiling_on_sc=False),
    )
    def gather(idx_hbm, table_hbm, out_hbm, idx_v, rows_v, sem):
        wid = lax.axis_index("s") * nc + lax.axis_index("c")
        base = wid * B_PER_W

        @pl.when(base < CTX_LEN)
        def _():
            pltpu.sync_copy(idx_hbm.at[pl.ds(base, B_PER_W)], idx_v)
            pltpu.async_copy(table_hbm.at[idx_v], rows_v, sem).wait()
            for r in range(B_PER_W):
                pltpu.sync_copy(
                    rows_v.at[pl.ds(r, 1)],
                    out_hbm.at[pl.ds(0, 1),
                               pl.ds((base + r) * EMBED_DIM, EMBED_DIM)],
                )

    return gather


def _mlp_body(embeds_ref, w1_ref, b1_ref, w2_ref, b2_ref, out_ref):
    h = lax.dot_general(
        embeds_ref[...], w1_ref[...],
        (((1,), (1,)), ((), ())),
        preferred_element_type=jnp.float32,
    )
    h = jnp.maximum(h + b1_ref[...], 0.0)
    logits = lax.dot_general(
        h, w2_ref[...],
        (((1,), (1,)), ((), ())),
        preferred_element_type=jnp.float32,
    ) + b2_ref[...]
    m = jnp.max(logits)
    s = jnp.sum(jnp.exp(logits - m))
    out_ref[...] = logits - m - jnp.log(s)


def _tc_mlp(embeds, w1, b1, w2, b2):
    return pl.pallas_call(
        _mlp_body,
        out_shape=jax.ShapeDtypeStruct((1, VOCAB), jnp.float32),
    )(embeds, w1, b1, w2, b2)


def kernel(context, emb_table, W1, b1, W2, b2):
    embeds = _make_sc_gather()(context.astype(jnp.int32), emb_table)
    return _tc_mlp(embeds, W1, b1.reshape(1, HIDDEN), W2,
                   b2.reshape(1, VOCAB))


# speedup vs baseline: 1.3826x; 1.3826x over previous
import jax
import jax.numpy as jnp
from jax import lax
from jax.experimental import pallas as pl
from jax.experimental.pallas import tpu as pltpu

VOCAB = 100000
EMBED_DIM = 64
CTX_LEN = 200
HIDDEN = 64
FLAT = CTX_LEN * EMBED_DIM

CHUNKS = [(0, 24960), (24960, 24960), (49920, 24960), (74880, 25120)]
CMAX = 25120


def _body(ctx_ref, table_ref, w1r_ref, b1_ref, w2_ref, b2_ref, out_ref,
          emb_scr, w1_scr, w2a_scr, w2b_scr, sem, w1_sem, w2a_sem, w2b_sem):
    w2_scr = [w2a_scr, w2b_scr]
    w2_sem = [w2a_sem, w2b_sem]

    def chunk_copy(k):
        off, w = CHUNKS[k]
        return pltpu.make_async_copy(
            w2_ref.at[pl.ds(off, w), :],
            w2_scr[k % 2].at[pl.ds(0, w), :],
            w2_sem[k % 2],
        )

    gathers = []
    for j in range(CTX_LEN):
        c = pltpu.make_async_copy(
            table_ref.at[pl.ds(ctx_ref[j], 1), :],
            emb_scr.at[pl.ds(j, 1), :],
            sem,
        )
        c.start()
        gathers.append(c)
    w1c = pltpu.make_async_copy(w1r_ref, w1_scr, w1_sem)
    w1c.start()
    pending = [chunk_copy(0), chunk_copy(1)]
    pending[0].start()
    pending[1].start()
    for c in gathers:
        c.wait()
    w1c.wait()

    h = b1_ref[...]
    for j in range(CTX_LEN):
        h = h + lax.dot_general(
            emb_scr[pl.ds(j, 1), :],
            w1_scr[pl.ds(j * EMBED_DIM, EMBED_DIM), :],
            (((1,), (0,)), ((), ())),
            preferred_element_type=jnp.float32,
        )
    h = jnp.maximum(h, 0.0)

    for k in range(len(CHUNKS)):
        off, w = CHUNKS[k]
        pending[k].wait()
        logits = lax.dot_general(
            h, w2_scr[k % 2][pl.ds(0, w), :],
            (((1,), (1,)), ((), ())),
            preferred_element_type=jnp.float32,
        ) + b2_ref[:, pl.ds(off, w)]
        out_ref[:, pl.ds(off, w)] = logits
        if k + 2 < len(CHUNKS):
            nxt = chunk_copy(k + 2)
            nxt.start()
            pending.append(nxt)

    l = out_ref[...]
    m = jnp.max(l)
    s = jnp.sum(jnp.exp(l - m))
    out_ref[...] = l - m - jnp.log(s)


def kernel(context, emb_table, W1, b1, W2, b2):
    W1r = W1.T
    return pl.pallas_call(
        _body,
        in_specs=[
            pl.BlockSpec(memory_space=pltpu.SMEM),
            pl.BlockSpec(memory_space=pl.ANY),
            pl.BlockSpec(memory_space=pl.ANY),
            pl.BlockSpec((1, HIDDEN), lambda: (0, 0)),
            pl.BlockSpec(memory_space=pl.ANY),
            pl.BlockSpec((1, VOCAB), lambda: (0, 0)),
        ],
        out_specs=pl.BlockSpec((1, VOCAB), lambda: (0, 0)),
        out_shape=jax.ShapeDtypeStruct((1, VOCAB), jnp.float32),
        scratch_shapes=[
            pltpu.VMEM((CTX_LEN, EMBED_DIM), jnp.float32),
            pltpu.VMEM((FLAT, HIDDEN), jnp.float32),
            pltpu.VMEM((CMAX, EMBED_DIM), jnp.float32),
            pltpu.VMEM((CMAX, EMBED_DIM), jnp.float32),
            pltpu.SemaphoreType.DMA,
            pltpu.SemaphoreType.DMA,
            pltpu.SemaphoreType.DMA,
            pltpu.SemaphoreType.DMA,
        ],
    )(context, emb_table, W1r, b1.reshape(1, HIDDEN), W2,
      b2.reshape(1, VOCAB))
